# baseline (speedup 1.0000x reference)
import functools

import jax
import jax.numpy as jnp
import numpy as np
from jax import lax
from jax.experimental import pallas as pl
from jax.experimental.pallas import tpu as pltpu
from jax.experimental.pallas import tpu_sc as plsc

P = 0.5
_B = 4
_ROWS = 2048
_COLS = 1024
_N = _ROWS * _COLS
_K = _N // 2
_SCALE = float(1.0 / np.float32(1.0 - P + 1e-12))

_NW = 8
_SLICE = _N // _NW
_CHUNK = 8192
_NCHUNK = _SLICE // _CHUNK
_VPC = _CHUNK // 16
_SC_PASSES = ((20, 2048, None), (10, 1024, 20), (0, 1024, 10))


_HSTAGE = 0
_PARTS = 524288
_SELP = 524544
_SELK = 524576
_SHWORDS = 524608


def _sc_threshold_body(grad_hbm, thr_hbm, buf, hist, tmp, acc, tot, sva,
                       svb, svc, pvm, fvec, shr):
    c = lax.axis_index("c")
    s = lax.axis_index("s")
    bgrp = s // _NW
    w = s % _NW
    b = c * 2 + bgrp
    lanes = lax.iota(jnp.int32, 16)
    ones = jnp.ones((16,), jnp.int32)
    zvec = jnp.zeros((16,), jnp.int32)

    k_cur = jnp.int32(_K)
    pref = jnp.int32(0)
    path = []

    for bin_shift, nb, filt_shift in _SC_PASSES:
        rng = nb // _NW

        def zbody(i, _):
            hist[pl.ds(i * 16, 16)] = zvec
            return 0

        lax.fori_loop(0, nb, zbody, 0)

        fpref = pref

        def chunk_body(g, _):
            start = w * _SLICE + g * _CHUNK
            pltpu.sync_copy(grad_hbm.at[b, pl.ds(start, _CHUNK)], buf)

            def vec_body(i, _):
                for u in range(8):
                    v = buf[pl.ds((i * 8 + u) * 16, 16)]
                    bits = lax.bitcast_convert_type(v, jnp.int32) & jnp.int32(
                        0x7FFFFFFF
                    )
                    bin_ = (bits >> bin_shift) & jnp.int32(nb - 1)
                    idx = (bin_ << 4) | lanes
                    if filt_shift is None:
                        plsc.addupdate_scatter(hist, [idx], ones)
                    else:
                        m = (bits >> filt_shift) == fpref
                        plsc.addupdate_scatter(hist, [idx], ones, mask=m)
                return 0

            lax.fori_loop(0, _VPC // 8, vec_body, 0)
            return 0

        lax.fori_loop(0, _NCHUNK, chunk_body, 0)

        slot = (bgrp * _NW + w) * 32768
        pltpu.sync_copy(hist.at[pl.ds(0, nb * 16)],
                        shr.at[pl.ds(_HSTAGE + slot, nb * 16)])
        plsc.subcore_barrier()

        def zacc(i, _):
            acc[pl.ds(i * 16, 16)] = zvec
            return 0

        lax.fori_loop(0, rng, zacc, 0)
        for src in range(_NW):
            soff = (bgrp * _NW + src) * 32768 + w * rng * 16
            pltpu.sync_copy(shr.at[pl.ds(_HSTAGE + soff, rng * 16)],
                            tmp.at[pl.ds(0, rng * 16)])

            def arow(i, _):
                acc[pl.ds(i * 16, 16)] = (
                    acc[pl.ds(i * 16, 16)] + tmp[pl.ds(i * 16, 16)]
                )
                return 0

            lax.fori_loop(0, rng, arow, 0)

        for blk in range(rng // 16):
            base = jnp.int32(blk * 256) + lanes * 16
            t = zvec
            for cidx in range(16):
                t = t + plsc.load_gather(acc, [base + jnp.int32(cidx)])
            tot[pl.ds(blk * 16, 16)] = t

        sv = zvec
        for i in range(rng // 16):
            sv = sv + tot[pl.ds(i * 16, 16)]
        s_w = jnp.sum(sv)
        sva[:] = jnp.full((16,), s_w, jnp.int32)
        pltpu.sync_copy(sva, shr.at[pl.ds(_PARTS + (bgrp * _NW + w) * 16, 16)])
        plsc.subcore_barrier()

        pltpu.sync_copy(shr.at[pl.ds(_PARTS + bgrp * _NW * 16, _NW * 16)], pvm)
        diag = plsc.load_gather(pvm, [jnp.minimum(lanes, _NW - 1) * 17])
        tvals = jnp.where(lanes < _NW, diag, 0)
        total = jnp.sum(tvals)
        cum = plsc.cumsum(tvals)
        suff = total - cum + tvals
        amask = jnp.logical_and(suff >= k_cur, lanes < _NW)
        wcnt = plsc.all_reduce_population_count(amask)
        wwin_v = wcnt - 1
        svb[:] = suff
        a_w_v = plsc.load_gather(svb, [wwin_v])
        svc[:] = tvals
        s_win_v = plsc.load_gather(svc, [wwin_v])
        above_w_v = a_w_v - s_win_v
        sva[:] = wwin_v
        w_win = sva[pl.ds(0, 16)][0]

        @pl.when(w == w_win)
        def _():
            def sbody(j, carry):
                suf_v, found_v, p_v, above_v = carry
                blk = rng // 16 - 1 - j
                v = tot[pl.ds(blk * 16, 16)]
                rv = lax.rev(v, (0,))
                cums = plsc.cumsum(rv)
                ex = cums - rv
                a_vals = above_w_v + suf_v + cums
                mask = a_vals >= k_cur
                lstar = plsc.all_reduce_ffs(mask)
                anyv = plsc.all_reduce_population_count(mask) > 0
                lidx = jnp.minimum(lstar, 15)
                svc[:] = ex
                exl = plsc.load_gather(svc, [lidx])
                binv = blk * 16 + 15 - lidx
                upd = jnp.logical_and(anyv, found_v == 0)
                p_v = jnp.where(upd, binv, p_v)
                above_v = jnp.where(upd, above_w_v + suf_v + exl, above_v)
                found_v = jnp.where(upd, 1, found_v)
                svc[:] = cums
                btot = plsc.load_gather(svc, [jnp.full((16,), 15, jnp.int32)])
                return (suf_v + btot, found_v, p_v, above_v)

            init = (zvec, zvec, zvec, zvec)
            _, _, p_v, above_v = lax.fori_loop(0, rng // 16, sbody, init)
            p_glob = wwin_v * jnp.int32(rng) + p_v
            k_next = k_cur - above_v
            sva[:] = p_glob
            pltpu.sync_copy(sva, shr.at[pl.ds(_SELP + bgrp * 16, 16)])
            svb[:] = k_next
            pltpu.sync_copy(svb, shr.at[pl.ds(_SELK + bgrp * 16, 16)])

        plsc.subcore_barrier()
        pltpu.sync_copy(shr.at[pl.ds(_SELP + bgrp * 16, 16)], sva)
        pltpu.sync_copy(shr.at[pl.ds(_SELK + bgrp * 16, 16)], svb)
        p_sel = sva[pl.ds(0, 16)][0]
        k_cur = svb[pl.ds(0, 16)][0]
        nbits = {2048: 11, 1024: 10}[nb]
        pref = (pref << nbits) | p_sel
        path.append(p_sel)

    tbits = (path[0] << 20) | (path[1] << 10) | path[2]
    fvec[:] = lax.bitcast_convert_type(
        jnp.full((16,), tbits, jnp.int32), jnp.float32
    )

    @pl.when(w == 0)
    def _():
        pltpu.sync_copy(fvec, thr_hbm.at[b])


_sc_threshold = pl.kernel(
    _sc_threshold_body,
    out_type=jax.ShapeDtypeStruct((_B, 16), jnp.float32),
    mesh=plsc.VectorSubcoreMesh(core_axis_name="c", subcore_axis_name="s"),
    compiler_params=pltpu.CompilerParams(needs_layout_passes=False),
    scratch_types=[
        pltpu.VMEM((_CHUNK,), jnp.float32),
        pltpu.VMEM((32768,), jnp.int32),
        pltpu.VMEM((4096,), jnp.int32),
        pltpu.VMEM((4096,), jnp.int32),
        pltpu.VMEM((256,), jnp.int32),
        pltpu.VMEM((16,), jnp.int32),
        pltpu.VMEM((16,), jnp.int32),
        pltpu.VMEM((16,), jnp.int32),
        pltpu.VMEM((_NW * 16,), jnp.int32),
        pltpu.VMEM((16,), jnp.float32),
        pltpu.VMEM_SHARED((_SHWORDS,), jnp.int32),
    ],
)


def _mask_kernel(thr_ref, x_ref, g_ref, o_ref):
    thr = thr_ref[0, 0, 0]
    mag = jnp.abs(g_ref[0])
    mask = (mag < thr).astype(jnp.float32)
    o_ref[0] = x_ref[0] * mask * _SCALE


@jax.jit
def kernel(x, grad):
    b = x.shape[0]
    thr16 = _sc_threshold(grad.reshape(b, -1))
    thr = thr16.reshape(b, 1, 16)

    rows_per_step = 512
    steps = _ROWS // rows_per_step
    out = pl.pallas_call(
        _mask_kernel,
        grid=(b, steps),
        in_specs=[
            pl.BlockSpec((1, 1, 16), lambda i, j: (i, 0, 0)),
            pl.BlockSpec((1, rows_per_step, _COLS), lambda i, j: (i, j, 0)),
            pl.BlockSpec((1, rows_per_step, _COLS), lambda i, j: (i, j, 0)),
        ],
        out_specs=pl.BlockSpec((1, rows_per_step, _COLS), lambda i, j: (i, j, 0)),
        out_shape=jax.ShapeDtypeStruct(x.shape, jnp.float32),
    )(thr, x, grad)
    return out


# speedup vs baseline: 1.5463x; 1.5463x over previous
import functools

import jax
import jax.numpy as jnp
import numpy as np
from jax import lax
from jax.experimental import pallas as pl
from jax.experimental.pallas import tpu as pltpu
from jax.experimental.pallas import tpu_sc as plsc

P = 0.5
_B = 4
_ROWS = 2048
_COLS = 1024
_N = _ROWS * _COLS
_K = _N // 2
_SCALE = float(1.0 / np.float32(1.0 - P + 1e-12))

_NW = 8
_SLICE = _N // _NW
_CHUNK = 8192
_NCHUNK = _SLICE // _CHUNK

_HSTAGE = 0
_PARTS = 524288
_SELP = 524544
_SELK = 524576
_SHWORDS = 524608


def _sc_threshold_body(grad_hbm, thr_hbm, buf0, buf1, hist, tmp, acc, sva,
                       svb, svc, pvm, fvec, shr, sem0, sem1):
    c = lax.axis_index("c")
    s = lax.axis_index("s")
    bgrp = s // _NW
    w = s % _NW
    b = c * 2 + bgrp
    lanes = lax.iota(jnp.int32, 16)
    ones = jnp.ones((16,), jnp.int32)
    zvec = jnp.zeros((16,), jnp.int32)

    k_cur = jnp.int32(_K)
    p1_sel = None
    path = []

    for pass_i, nb in ((0, 65536), (1, 32768)):
        rng = nb // _NW
        nblk = rng // 16
        base = w * _SLICE

        def fire(g, dstbuf, dstsem):
            pltpu.async_copy(grad_hbm.at[b, pl.ds(base + g * _CHUNK, _CHUNK)],
                             dstbuf, dstsem)

        def wait(g, dstbuf, dstsem):
            pltpu.make_async_copy(
                grad_hbm.at[b, pl.ds(base + g * _CHUNK, _CHUNK)],
                dstbuf, dstsem).wait()

        fire(0, buf0, sem0)
        fire(1, buf1, sem1)

        def zbody(i, _):
            for u in range(8):
                hist[pl.ds((i * 8 + u) * 16, 16)] = zvec
            return 0

        lax.fori_loop(0, nb // 128, zbody, 0)

        fpref = p1_sel

        def proc(bufref):
            def vbody(i, _):
                for u in range(8):
                    v = bufref[pl.ds((i * 8 + u) * 16, 16)]
                    bc = lax.bitcast_convert_type(v, jnp.int32)
                    if pass_i == 0:
                        idx = (bc >> 15) & jnp.int32(0xFFFF)
                        plsc.addupdate_scatter(hist, [idx], ones)
                    else:
                        idx = bc & jnp.int32(0x7FFF)
                        m = ((bc >> 15) & jnp.int32(0xFFFF)) == fpref
                        plsc.addupdate_scatter(hist, [idx], ones, mask=m)
                return 0

            lax.fori_loop(0, _CHUNK // 128, vbody, 0)

        def chunk_pair(i, _):
            g0 = 2 * i
            wait(g0, buf0, sem0)
            proc(buf0)

            @pl.when(g0 + 2 < _NCHUNK)
            def _():
                fire(g0 + 2, buf0, sem0)

            wait(g0 + 1, buf1, sem1)
            proc(buf1)

            @pl.when(g0 + 3 < _NCHUNK)
            def _():
                fire(g0 + 3, buf1, sem1)

            return 0

        lax.fori_loop(0, _NCHUNK // 2, chunk_pair, 0)

        slot = (bgrp * _NW + w) * 32768
        half_nb = nb // 2
        for half in range(2):
            pltpu.sync_copy(hist.at[pl.ds(half * half_nb, half_nb)],
                            shr.at[pl.ds(_HSTAGE + slot, half_nb)])
            plsc.subcore_barrier()

            @pl.when((w // 4) == half)
            def _():
                def zacc(i, _):
                    for u in range(4):
                        acc[pl.ds((i * 4 + u) * 16, 16)] = zvec
                    return 0

                lax.fori_loop(0, nblk // 4, zacc, 0)
                for src in range(_NW):
                    for q in range(max(1, rng // 4096)):
                        qn = min(rng, 4096)
                        soff = ((bgrp * _NW + src) * 32768
                                + (w % 4) * rng + q * qn)
                        pltpu.sync_copy(shr.at[pl.ds(_HSTAGE + soff, qn)],
                                        tmp.at[pl.ds(0, qn)])

                        def arow(i, _):
                            for u in range(4):
                                o = (i * 4 + u) * 16
                                acc[pl.ds(q * qn + o, 16)] = (
                                    acc[pl.ds(q * qn + o, 16)]
                                    + tmp[pl.ds(o, 16)]
                                )
                            return 0

                        lax.fori_loop(0, qn // 64, arow, 0)

            plsc.subcore_barrier()

        def sbody(i, sv):
            for u in range(4):
                sv = sv + acc[pl.ds((i * 4 + u) * 16, 16)]
            return sv

        sv = lax.fori_loop(0, nblk // 4, sbody, zvec)
        s_w = jnp.sum(sv)
        sva[:] = jnp.full((16,), s_w, jnp.int32)
        pltpu.sync_copy(sva, shr.at[pl.ds(_PARTS + (bgrp * _NW + w) * 16, 16)])
        plsc.subcore_barrier()

        pltpu.sync_copy(shr.at[pl.ds(_PARTS + bgrp * _NW * 16, _NW * 16)], pvm)
        diag = plsc.load_gather(pvm, [jnp.minimum(lanes, _NW - 1) * 17])
        tvals = jnp.where(lanes < _NW, diag, 0)
        total = jnp.sum(tvals)
        cum = plsc.cumsum(tvals)
        suff = total - cum + tvals
        amask = jnp.logical_and(suff >= k_cur, lanes < _NW)
        wcnt = plsc.all_reduce_population_count(amask)
        wwin_v = wcnt - 1
        svb[:] = suff
        a_w_v = plsc.load_gather(svb, [wwin_v])
        svc[:] = tvals
        s_win_v = plsc.load_gather(svc, [wwin_v])
        above_w_v = a_w_v - s_win_v
        w_win = wwin_v[0]

        @pl.when(w == w_win)
        def _():
            def wcond(carry):
                j, found, _, _, _ = carry
                return jnp.logical_and(found == 0, j < nblk)

            def wbody(carry):
                j, found, p_v, above_v, suf_v = carry
                blk = jnp.int32(nblk - 1) - j
                v = acc[pl.ds(blk * 16, 16)]
                rv = lax.rev(v, (0,))
                cums = plsc.cumsum(rv)
                a_vals = above_w_v + suf_v + cums
                mask = a_vals >= k_cur
                lstar = plsc.all_reduce_ffs(mask)
                any_s = plsc.all_reduce_population_count(mask)[0] > 0
                lidx = jnp.minimum(lstar, 15)
                svc[:] = cums
                cl = plsc.load_gather(svc, [lidx])
                btot = plsc.load_gather(svc, [jnp.full((16,), 15, jnp.int32)])
                svb[:] = rv
                rl = plsc.load_gather(svb, [lidx])
                binv = blk * 16 + 15 - lidx
                p_v = jnp.where(any_s, binv, p_v)
                above_v = jnp.where(any_s, above_w_v + suf_v + (cl - rl),
                                    above_v)
                found = jnp.where(any_s, jnp.int32(1), found)
                return (j + 1, found, p_v, above_v, suf_v + btot)

            init = (jnp.int32(0), jnp.int32(0), zvec, zvec, zvec)
            _, _, p_v, above_v, _ = lax.while_loop(wcond, wbody, init)
            p_glob = wwin_v * jnp.int32(rng) + p_v
            k_next = k_cur - above_v
            sva[:] = p_glob
            pltpu.sync_copy(sva, shr.at[pl.ds(_SELP + bgrp * 16, 16)])
            svb[:] = k_next
            pltpu.sync_copy(svb, shr.at[pl.ds(_SELK + bgrp * 16, 16)])

        plsc.subcore_barrier()
        pltpu.sync_copy(shr.at[pl.ds(_SELP + bgrp * 16, 16)], sva)
        pltpu.sync_copy(shr.at[pl.ds(_SELK + bgrp * 16, 16)], svb)
        p_sel = sva[pl.ds(0, 16)][0]
        k_cur = svb[pl.ds(0, 16)][0]
        if pass_i == 0:
            p1_sel = p_sel
        path.append(p_sel)

    tbits = (path[0] << 15) | path[1]
    fvec[:] = lax.bitcast_convert_type(
        jnp.full((16,), tbits, jnp.int32), jnp.float32
    )

    @pl.when(w == 0)
    def _():
        pltpu.sync_copy(fvec, thr_hbm.at[b])


_sc_threshold = pl.kernel(
    _sc_threshold_body,
    out_type=jax.ShapeDtypeStruct((_B, 16), jnp.float32),
    mesh=plsc.VectorSubcoreMesh(core_axis_name="c", subcore_axis_name="s"),
    compiler_params=pltpu.CompilerParams(needs_layout_passes=False),
    scratch_types=[
        pltpu.VMEM((_CHUNK,), jnp.float32),
        pltpu.VMEM((_CHUNK,), jnp.float32),
        pltpu.VMEM((65536,), jnp.int32),
        pltpu.VMEM((4096,), jnp.int32),
        pltpu.VMEM((8192,), jnp.int32),
        pltpu.VMEM((16,), jnp.int32),
        pltpu.VMEM((16,), jnp.int32),
        pltpu.VMEM((16,), jnp.int32),
        pltpu.VMEM((128,), jnp.int32),
        pltpu.VMEM((16,), jnp.float32),
        pltpu.VMEM_SHARED((_SHWORDS,), jnp.int32),
        pltpu.SemaphoreType.DMA,
        pltpu.SemaphoreType.DMA,
    ],
)


def _mask_kernel(thr_ref, x_ref, g_ref, o_ref):
    thr = thr_ref[0, 0, 0]
    mag = jnp.abs(g_ref[0])
    mask = (mag < thr).astype(jnp.float32)
    o_ref[0] = x_ref[0] * mask * _SCALE


@jax.jit
def kernel(x, grad):
    b = x.shape[0]
    thr16 = _sc_threshold(grad.reshape(b, -1))
    thr = thr16.reshape(b, 1, 16)

    rows_per_step = 512
    steps = _ROWS // rows_per_step
    out = pl.pallas_call(
        _mask_kernel,
        grid=(b, steps),
        in_specs=[
            pl.BlockSpec((1, 1, 16), lambda i, j: (i, 0, 0)),
            pl.BlockSpec((1, rows_per_step, _COLS), lambda i, j: (i, j, 0)),
            pl.BlockSpec((1, rows_per_step, _COLS), lambda i, j: (i, j, 0)),
        ],
        out_specs=pl.BlockSpec((1, rows_per_step, _COLS), lambda i, j: (i, j, 0)),
        out_shape=jax.ShapeDtypeStruct(x.shape, jnp.float32),
    )(thr, x, grad)
    return out


# speedup vs baseline: 3.4052x; 2.2021x over previous
import functools

import jax
import jax.numpy as jnp
import numpy as np
from jax import lax
from jax.experimental import pallas as pl
from jax.experimental.pallas import tpu as pltpu
from jax.experimental.pallas import tpu_sc as plsc

P = 0.5
_B = 4
_ROWS = 2048
_COLS = 1024
_N = _ROWS * _COLS
_K = _N // 2
_SCALE = float(1.0 / np.float32(1.0 - P + 1e-12))

_NW = 8
_SLICE = _N // _NW
_CHUNK = 8192
_NCHUNK = _SLICE // _CHUNK

_HSTAGE = 0
_PARTS = 524288
_SELP = 524544
_SELK = 524576
_SHWORDS = 524608


def _sc_threshold_body(grad_hbm, thr_hbm, buf0, buf1, hist, tmp, acc, sva,
                       svb, svc, pvm, fvec, shr, sem0, sem1):
    c = lax.axis_index("c")
    s = lax.axis_index("s")
    bgrp = s // _NW
    w = s % _NW
    b = c * 2 + bgrp
    lanes = lax.iota(jnp.int32, 16)
    ones = jnp.ones((16,), jnp.int32)
    zvec = jnp.zeros((16,), jnp.int32)

    k_cur = jnp.int32(_K)
    p1_sel = None
    path = []

    for pass_i, nb in ((0, 65536), (1, 32768)):
        rng = nb // _NW
        nblk = rng // 16
        base = w * _SLICE

        def fire(g, dstbuf, dstsem):
            pltpu.async_copy(grad_hbm.at[b, pl.ds(base + g * _CHUNK, _CHUNK)],
                             dstbuf, dstsem)

        def wait(g, dstbuf, dstsem):
            pltpu.make_async_copy(
                grad_hbm.at[b, pl.ds(base + g * _CHUNK, _CHUNK)],
                dstbuf, dstsem).wait()

        fire(0, buf0, sem0)
        fire(1, buf1, sem1)

        @plsc.parallel_loop(0, nb // 16, 1, unroll=8)
        def _(i):
            hist[pl.ds(i * 16, 16)] = zvec

        fpref = p1_sel

        def proc(bufref):
            @plsc.parallel_loop(0, _CHUNK // 16, 1, unroll=8)
            def _(i):
                v = bufref[pl.ds(i * 16, 16)]
                bc = lax.bitcast_convert_type(v, jnp.int32)
                if pass_i == 0:
                    idx = (bc >> 15) & jnp.int32(0xFFFF)
                    plsc.addupdate_scatter(hist, [idx], ones)
                else:
                    idx = bc & jnp.int32(0x7FFF)
                    m = ((bc >> 15) & jnp.int32(0xFFFF)) == fpref
                    plsc.addupdate_scatter(hist, [idx], ones, mask=m)

        def chunk_pair(i, _):
            g0 = 2 * i
            wait(g0, buf0, sem0)
            proc(buf0)

            @pl.when(g0 + 2 < _NCHUNK)
            def _():
                fire(g0 + 2, buf0, sem0)

            wait(g0 + 1, buf1, sem1)
            proc(buf1)

            @pl.when(g0 + 3 < _NCHUNK)
            def _():
                fire(g0 + 3, buf1, sem1)

            return 0

        lax.fori_loop(0, _NCHUNK // 2, chunk_pair, 0)

        slot = (bgrp * _NW + w) * 32768
        half_nb = nb // 2
        for half in range(2):
            pltpu.sync_copy(hist.at[pl.ds(half * half_nb, half_nb)],
                            shr.at[pl.ds(_HSTAGE + slot, half_nb)])
            plsc.subcore_barrier()

            @pl.when((w // 4) == half)
            def _():
                @plsc.parallel_loop(0, nblk, 1, unroll=4)
                def _(i):
                    acc[pl.ds(i * 16, 16)] = zvec

                for src in range(_NW):
                    for q in range(max(1, rng // 4096)):
                        qn = min(rng, 4096)
                        soff = ((bgrp * _NW + src) * 32768
                                + (w % 4) * rng + q * qn)
                        pltpu.sync_copy(shr.at[pl.ds(_HSTAGE + soff, qn)],
                                        tmp.at[pl.ds(0, qn)])

                        @plsc.parallel_loop(0, qn // 16, 1, unroll=4)
                        def _(i):
                            o = i * 16
                            acc[pl.ds(q * qn + o, 16)] = (
                                acc[pl.ds(q * qn + o, 16)] + tmp[pl.ds(o, 16)]
                            )

            plsc.subcore_barrier()

        @plsc.parallel_loop(0, nblk, 1, unroll=4, carry=zvec)
        def sv(i, cv):
            return cv + acc[pl.ds(i * 16, 16)]

        s_w = jnp.sum(sv)
        sva[:] = jnp.full((16,), s_w, jnp.int32)
        pltpu.sync_copy(sva, shr.at[pl.ds(_PARTS + (bgrp * _NW + w) * 16, 16)])
        plsc.subcore_barrier()

        pltpu.sync_copy(shr.at[pl.ds(_PARTS + bgrp * _NW * 16, _NW * 16)], pvm)
        diag = plsc.load_gather(pvm, [jnp.minimum(lanes, _NW - 1) * 17])
        tvals = jnp.where(lanes < _NW, diag, 0)
        total = jnp.sum(tvals)
        cum = plsc.cumsum(tvals)
        suff = total - cum + tvals
        amask = jnp.logical_and(suff >= k_cur, lanes < _NW)
        wcnt = plsc.all_reduce_population_count(amask)
        wwin_v = wcnt - 1
        svb[:] = suff
        a_w_v = plsc.load_gather(svb, [wwin_v])
        svc[:] = tvals
        s_win_v = plsc.load_gather(svc, [wwin_v])
        above_w_v = a_w_v - s_win_v
        w_win = wwin_v[0]

        @pl.when(w == w_win)
        def _():
            def wcond(carry):
                j, found, _, _, _ = carry
                return jnp.logical_and(found == 0, j < nblk)

            def wbody(carry):
                j, found, p_v, above_v, suf_v = carry
                blk = jnp.int32(nblk - 1) - j
                v = acc[pl.ds(blk * 16, 16)]
                rv = lax.rev(v, (0,))
                cums = plsc.cumsum(rv)
                a_vals = above_w_v + suf_v + cums
                mask = a_vals >= k_cur
                lstar = plsc.all_reduce_ffs(mask)
                any_s = plsc.all_reduce_population_count(mask)[0] > 0
                lidx = jnp.minimum(lstar, 15)
                svc[:] = cums
                cl = plsc.load_gather(svc, [lidx])
                btot = plsc.load_gather(svc, [jnp.full((16,), 15, jnp.int32)])
                svb[:] = rv
                rl = plsc.load_gather(svb, [lidx])
                binv = blk * 16 + 15 - lidx
                p_v = jnp.where(any_s, binv, p_v)
                above_v = jnp.where(any_s, above_w_v + suf_v + (cl - rl),
                                    above_v)
                found = jnp.where(any_s, jnp.int32(1), found)
                return (j + 1, found, p_v, above_v, suf_v + btot)

            init = (jnp.int32(0), jnp.int32(0), zvec, zvec, zvec)
            _, _, p_v, above_v, _ = lax.while_loop(wcond, wbody, init)
            p_glob = wwin_v * jnp.int32(rng) + p_v
            k_next = k_cur - above_v
            sva[:] = p_glob
            pltpu.sync_copy(sva, shr.at[pl.ds(_SELP + bgrp * 16, 16)])
            svb[:] = k_next
            pltpu.sync_copy(svb, shr.at[pl.ds(_SELK + bgrp * 16, 16)])

        plsc.subcore_barrier()
        pltpu.sync_copy(shr.at[pl.ds(_SELP + bgrp * 16, 16)], sva)
        pltpu.sync_copy(shr.at[pl.ds(_SELK + bgrp * 16, 16)], svb)
        p_sel = sva[pl.ds(0, 16)][0]
        k_cur = svb[pl.ds(0, 16)][0]
        if pass_i == 0:
            p1_sel = p_sel
        path.append(p_sel)

    tbits = (path[0] << 15) | path[1]
    fvec[:] = lax.bitcast_convert_type(
        jnp.full((16,), tbits, jnp.int32), jnp.float32
    )

    @pl.when(w == 0)
    def _():
        pltpu.sync_copy(fvec, thr_hbm.at[b])


_sc_threshold = pl.kernel(
    _sc_threshold_body,
    out_type=jax.ShapeDtypeStruct((_B, 16), jnp.float32),
    mesh=plsc.VectorSubcoreMesh(core_axis_name="c", subcore_axis_name="s"),
    compiler_params=pltpu.CompilerParams(needs_layout_passes=False),
    scratch_types=[
        pltpu.VMEM((_CHUNK,), jnp.float32),
        pltpu.VMEM((_CHUNK,), jnp.float32),
        pltpu.VMEM((65536,), jnp.int32),
        pltpu.VMEM((4096,), jnp.int32),
        pltpu.VMEM((8192,), jnp.int32),
        pltpu.VMEM((16,), jnp.int32),
        pltpu.VMEM((16,), jnp.int32),
        pltpu.VMEM((16,), jnp.int32),
        pltpu.VMEM((128,), jnp.int32),
        pltpu.VMEM((16,), jnp.float32),
        pltpu.VMEM_SHARED((_SHWORDS,), jnp.int32),
        pltpu.SemaphoreType.DMA,
        pltpu.SemaphoreType.DMA,
    ],
)


def _mask_kernel(thr_ref, x_ref, g_ref, o_ref):
    thr = thr_ref[0, 0, 0]
    mag = jnp.abs(g_ref[0])
    mask = (mag < thr).astype(jnp.float32)
    o_ref[0] = x_ref[0] * mask * _SCALE


@jax.jit
def kernel(x, grad):
    b = x.shape[0]
    thr16 = _sc_threshold(grad.reshape(b, -1))
    thr = thr16.reshape(b, 1, 16)

    rows_per_step = 512
    steps = _ROWS // rows_per_step
    out = pl.pallas_call(
        _mask_kernel,
        grid=(b, steps),
        in_specs=[
            pl.BlockSpec((1, 1, 16), lambda i, j: (i, 0, 0)),
            pl.BlockSpec((1, rows_per_step, _COLS), lambda i, j: (i, j, 0)),
            pl.BlockSpec((1, rows_per_step, _COLS), lambda i, j: (i, j, 0)),
        ],
        out_specs=pl.BlockSpec((1, rows_per_step, _COLS), lambda i, j: (i, j, 0)),
        out_shape=jax.ShapeDtypeStruct(x.shape, jnp.float32),
    )(thr, x, grad)
    return out
